# baseline (speedup 1.0000x reference)
import jax
import jax.numpy as jnp
import numpy as np
from jax import lax
from jax.experimental import pallas as pl
from jax.experimental.pallas import tpu as pltpu
from jax.experimental.pallas import tpu_sc as plsc

N_ROWS = 1000000
DIM = 16
B = 16384
L = 50

NC = 2
NS = 16
NW = NC * NS

CW = 512
NCHK = 999936 // CW
TAIL = N_ROWS - NCHK * CW
NJ = 62

UPW = B // NW
CU = 64
NCHUNK = UPW // CU
CI = CU * L
NG = CI // DIM


def _transpose_chunk(inv, outv, ncols, bc, d8c):
    def rows8(c8, _):
        for u in range(8):
            c = c8 * 8 + u
            pos = jnp.full((DIM,), 0, jnp.int32) + c
            row = plsc.load_gather(inv, [bc, d8c, pos])
            outv[pl.ds(c * DIM, DIM)] = row
        return 0
    lax.fori_loop(0, ncols // 8, rows8, 0)


def _fmt_body(it8, ut8, itF, utF,
              iiA, ioA, uiA, uoA, iiB, ioB, uiB, uoB,
              s_iiA, s_ioA, s_uiA, s_uoA, s_iiB, s_ioB, s_uiB, s_uoB):
    wid = lax.axis_index("s") * NC + lax.axis_index("c")
    iota = lax.iota(jnp.int32, DIM)
    bc = iota >> 3
    d8c = iota & 7

    def cid_of(j):
        return jnp.minimum(wid + j * NW, NCHK - 1)

    def issue_in(tab, inv, sem, j):
        c0 = pl.multiple_of(cid_of(j) * CW, CW)
        return pltpu.async_copy(tab.at[:, :, pl.ds(c0, CW)], inv, sem)

    def body(k2, _):
        j0 = 2 * k2
        j1 = 2 * k2 + 1
        dAi = issue_in(it8, iiA, s_iiA, j0)
        dAu = issue_in(ut8, uiA, s_uiA, j0)
        dBi = issue_in(it8, iiB, s_iiB, j1)
        dBu = issue_in(ut8, uiB, s_uiB, j1)
        for inv, outv, din, so, dstF, j in (
                (iiA, ioA, dAi, s_ioA, itF, j0),
                (uiA, uoA, dAu, s_uoA, utF, j0),
                (iiB, ioB, dBi, s_ioB, itF, j1),
                (uiB, uoB, dBu, s_uoB, utF, j1)):
            @pl.when(k2 > 0)
            def _():
                pltpu.make_async_copy(outv, dstF.at[pl.ds(0, CW * DIM)], so).wait()
            din.wait()
            _transpose_chunk(inv, outv, CW, bc, d8c)
            o0 = pl.multiple_of(cid_of(j) * (CW * DIM), CW * DIM)
            pltpu.async_copy(outv, dstF.at[pl.ds(o0, CW * DIM)], so)
        return 0

    lax.fori_loop(0, NJ // 2, body, 0)
    for outv, so, dstF in ((ioA, s_ioA, itF), (uoA, s_uoA, utF),
                           (ioB, s_ioB, itF), (uoB, s_uoB, utF)):
        pltpu.make_async_copy(outv, dstF.at[pl.ds(0, CW * DIM)], so).wait()


def _gmf_body(uidx_hbm, iidx_hbm, utab_hbm, itab_hbm, w_hbm, bias_hbm, uidl_hbm,
              out_hbm,
              uidl_v, wv, biasv,
              uidx_v0, iidx_v0, urows_v0, irows_v0, out_v0,
              uidx_v1, iidx_v1, urows_v1, irows_v1, out_v1,
              sem_g0, sem_g1, sem_o0, sem_o1):
    wid = lax.axis_index("s") * NC + lax.axis_index("c")

    pltpu.sync_copy(w_hbm, wv)
    pltpu.sync_copy(bias_hbm, biasv)
    pltpu.sync_copy(uidl_hbm, uidl_v)

    iota = lax.iota(jnp.int32, DIM)
    fds = [jnp.full((DIM,), d, dtype=jnp.int32) for d in range(DIM)]
    wsplat = [wv[d, :] for d in range(DIM)]
    bias_vec = biasv[...]

    bufs = [
        (uidx_v0, iidx_v0, urows_v0, irows_v0, out_v0, sem_g0, sem_o0),
        (uidx_v1, iidx_v1, urows_v1, irows_v1, out_v1, sem_g1, sem_o1),
    ]

    def issue(c):
        uidx_v, iidx_v, urows_v, irows_v, _, sg, _ = bufs[c % 2]
        ubase = wid * UPW + c * CU
        pltpu.sync_copy(uidx_hbm.at[pl.ds(ubase, CU)], uidx_v)
        pltpu.sync_copy(iidx_hbm.at[pl.ds(ubase * L, CI)], iidx_v)
        cu = pltpu.async_copy(utab_hbm.at[uidx_v], urows_v, sg)
        ci = pltpu.async_copy(itab_hbm.at[iidx_v], irows_v, sg)
        return cu, ci

    pending = issue(0)
    out_pending = [None, None]
    for c in range(NCHUNK):
        slot = c % 2
        _, _, urows_v, irows_v, out_v, _, so = bufs[slot]
        nxt = issue(c + 1) if c + 1 < NCHUNK else None
        pending[0].wait()
        pending[1].wait()
        if out_pending[slot] is not None:
            out_pending[slot].wait()
            out_pending[slot] = None

        def group_body(g, _, irows_v=irows_v, urows_v=urows_v, out_v=out_v):
            gb = pl.multiple_of(g * DIM, DIM)
            pos = gb + iota
            uid = uidl_v[pl.ds(gb, DIM)]
            acc = bias_vec
            for d in range(DIM):
                col = plsc.load_gather(irows_v, [pos, fds[d]])
                pd = plsc.load_gather(urows_v, [uid, fds[d]])
                acc = acc + col * pd * wsplat[d]
            out_v[pl.ds(gb, DIM)] = acc
            return 0

        lax.fori_loop(0, NG, group_body, 0)
        ibase = wid * UPW * L + c * CI
        out_pending[slot] = pltpu.async_copy(
            out_v, out_hbm.at[pl.ds(ibase, CI)], so)
        pending = nxt

    for slot in range(2):
        if out_pending[slot] is not None:
            out_pending[slot].wait()


@jax.jit
def _run(user_indices, item_indices, user_table, item_table, W, b):
    mesh = plsc.VectorSubcoreMesh(core_axis_name="c", subcore_axis_name="s")
    it8 = item_table.T.reshape(2, 8, N_ROWS)
    ut8 = user_table.T.reshape(2, 8, N_ROWS)

    fmt = pl.kernel(
        _fmt_body,
        out_type=(jax.ShapeDtypeStruct((N_ROWS * DIM,), jnp.float32),
                  jax.ShapeDtypeStruct((N_ROWS * DIM,), jnp.float32)),
        mesh=mesh,
        compiler_params=pltpu.CompilerParams(
            needs_layout_passes=False, use_tc_tiling_on_sc=True),
        scratch_types=(
            [pltpu.VMEM((2, 8, CW), jnp.float32), pltpu.VMEM((CW * DIM,), jnp.float32)] * 4
            + [pltpu.SemaphoreType.DMA] * 8),
    )
    itF, utF = fmt(it8, ut8)
    t0 = NCHK * CW
    itF = lax.dynamic_update_slice(itF, item_table[t0:].reshape(-1), (t0 * DIM,))
    utF = lax.dynamic_update_slice(utF, user_table[t0:].reshape(-1), (t0 * DIM,))

    item_idx_flat = item_indices.reshape(B * L)
    w16 = jnp.broadcast_to(W.reshape(DIM, 1), (DIM, DIM))
    bias16 = jnp.broadcast_to(b, (DIM,))
    uidl = jnp.asarray(np.arange(CI, dtype=np.int32) // L)

    gmf = pl.kernel(
        _gmf_body,
        out_type=jax.ShapeDtypeStruct((B * L,), jnp.float32),
        mesh=mesh,
        compiler_params=pltpu.CompilerParams(
            needs_layout_passes=False, use_tc_tiling_on_sc=False),
        scratch_types=(
            [pltpu.VMEM((CI,), jnp.int32),
             pltpu.VMEM((DIM, DIM), jnp.float32),
             pltpu.VMEM((DIM,), jnp.float32)]
            + [pltpu.VMEM((CU,), jnp.int32),
               pltpu.VMEM((CI,), jnp.int32),
               pltpu.VMEM((CU, DIM), jnp.float32),
               pltpu.VMEM((CI, DIM), jnp.float32),
               pltpu.VMEM((CI,), jnp.float32)] * 2
            + [pltpu.SemaphoreType.DMA] * 4),
    )
    out = gmf(user_indices, item_idx_flat,
              utF.reshape(N_ROWS, DIM), itF.reshape(N_ROWS, DIM),
              w16, bias16, uidl)
    return out.reshape(B, L, 1)


def kernel(user_indices, item_indices, user_table, item_table, W, b):
    return _run(user_indices, item_indices, user_table, item_table, W, b)


# speedup vs baseline: 2.2191x; 2.2191x over previous
import jax
import jax.numpy as jnp
import numpy as np
from jax import lax
from jax.experimental import pallas as pl
from jax.experimental.pallas import tpu as pltpu
from jax.experimental.pallas import tpu_sc as plsc

N_ROWS = 1000000
DIM = 16
B = 16384
L = 50

NC = 2
NS = 16
NW = NC * NS

CW = 512
NCHK = 999936 // CW
TAIL = N_ROWS - NCHK * CW
NJ = 62

UPW = B // NW
CU = 64
NCHUNK = UPW // CU
CI = CU * L
NG = CI // DIM


def _transpose_chunk(inv, outv, ncols, iota16):
    sc16 = iota16 * DIM

    def blk(c16, _):
        cbase = c16 * DIM
        for b in range(2):
            for d8 in range(8):
                vec = inv[b, d8, pl.ds(cbase, DIM)]
                idx = sc16 + (cbase * DIM + b * 8 + d8)
                plsc.store_scatter(outv, [idx], vec)
        return 0
    lax.fori_loop(0, ncols // DIM, blk, 0)


def _fmt_body(it8, ut8, itF, utF,
              iiA, ioA, uiA, uoA, iiB, ioB, uiB, uoB,
              s_iiA, s_ioA, s_uiA, s_uoA, s_iiB, s_ioB, s_uiB, s_uoB):
    wid = lax.axis_index("s") * NC + lax.axis_index("c")
    iota16 = lax.iota(jnp.int32, DIM)

    def cid_of(j):
        return jnp.minimum(wid + j * NW, NCHK - 1)

    def issue_in(tab, inv, sem, j):
        c0 = pl.multiple_of(cid_of(j) * CW, CW)
        return pltpu.async_copy(tab.at[:, :, pl.ds(c0, CW)], inv, sem)

    def body(k2, _):
        j0 = 2 * k2
        j1 = 2 * k2 + 1
        dAi = issue_in(it8, iiA, s_iiA, j0)
        dAu = issue_in(ut8, uiA, s_uiA, j0)
        dBi = issue_in(it8, iiB, s_iiB, j1)
        dBu = issue_in(ut8, uiB, s_uiB, j1)
        for inv, outv, din, so, dstF, j in (
                (iiA, ioA, dAi, s_ioA, itF, j0),
                (uiA, uoA, dAu, s_uoA, utF, j0),
                (iiB, ioB, dBi, s_ioB, itF, j1),
                (uiB, uoB, dBu, s_uoB, utF, j1)):
            @pl.when(k2 > 0)
            def _():
                pltpu.make_async_copy(outv, dstF.at[pl.ds(0, CW * DIM)], so).wait()
            din.wait()
            _transpose_chunk(inv, outv, CW, iota16)
            o0 = pl.multiple_of(cid_of(j) * (CW * DIM), CW * DIM)
            pltpu.async_copy(outv, dstF.at[pl.ds(o0, CW * DIM)], so)
        return 0

    lax.fori_loop(0, NJ // 2, body, 0)
    for outv, so, dstF in ((ioA, s_ioA, itF), (uoA, s_uoA, utF),
                           (ioB, s_ioB, itF), (uoB, s_uoB, utF)):
        pltpu.make_async_copy(outv, dstF.at[pl.ds(0, CW * DIM)], so).wait()


def _gmf_body(uidx_hbm, iidx_hbm, utab_hbm, itab_hbm, w_hbm, bias_hbm, uidl_hbm,
              out_hbm,
              uidl_v, wv, biasv,
              uidx_v0, iidx_v0, urows_v0, irows_v0, out_v0,
              uidx_v1, iidx_v1, urows_v1, irows_v1, out_v1,
              sem_g0, sem_g1, sem_o0, sem_o1):
    wid = lax.axis_index("s") * NC + lax.axis_index("c")

    pltpu.sync_copy(w_hbm, wv)
    pltpu.sync_copy(bias_hbm, biasv)
    pltpu.sync_copy(uidl_hbm, uidl_v)

    iota = lax.iota(jnp.int32, DIM)
    fds = [jnp.full((DIM,), d, dtype=jnp.int32) for d in range(DIM)]
    wsplat = [wv[d, :] for d in range(DIM)]
    bias_vec = biasv[...]

    bufs = [
        (uidx_v0, iidx_v0, urows_v0, irows_v0, out_v0, sem_g0, sem_o0),
        (uidx_v1, iidx_v1, urows_v1, irows_v1, out_v1, sem_g1, sem_o1),
    ]

    def issue(c):
        uidx_v, iidx_v, urows_v, irows_v, _, sg, _ = bufs[c % 2]
        ubase = wid * UPW + c * CU
        pltpu.sync_copy(uidx_hbm.at[pl.ds(ubase, CU)], uidx_v)
        pltpu.sync_copy(iidx_hbm.at[pl.ds(ubase * L, CI)], iidx_v)
        cu = pltpu.async_copy(utab_hbm.at[uidx_v], urows_v, sg)
        ci = pltpu.async_copy(itab_hbm.at[iidx_v], irows_v, sg)
        return cu, ci

    pending = issue(0)
    out_pending = [None, None]
    for c in range(NCHUNK):
        slot = c % 2
        _, _, urows_v, irows_v, out_v, _, so = bufs[slot]
        nxt = issue(c + 1) if c + 1 < NCHUNK else None
        pending[0].wait()
        pending[1].wait()
        if out_pending[slot] is not None:
            out_pending[slot].wait()
            out_pending[slot] = None

        def group_body(g, _, irows_v=irows_v, urows_v=urows_v, out_v=out_v):
            gb = pl.multiple_of(g * DIM, DIM)
            pos = gb + iota
            uid = uidl_v[pl.ds(gb, DIM)]
            acc = bias_vec
            for d in range(DIM):
                col = plsc.load_gather(irows_v, [pos, fds[d]])
                pd = plsc.load_gather(urows_v, [uid, fds[d]])
                acc = acc + col * pd * wsplat[d]
            out_v[pl.ds(gb, DIM)] = acc
            return 0

        lax.fori_loop(0, NG, group_body, 0)
        ibase = wid * UPW * L + c * CI
        out_pending[slot] = pltpu.async_copy(
            out_v, out_hbm.at[pl.ds(ibase, CI)], so)
        pending = nxt

    for slot in range(2):
        if out_pending[slot] is not None:
            out_pending[slot].wait()


@jax.jit
def _run(user_indices, item_indices, user_table, item_table, W, b):
    mesh = plsc.VectorSubcoreMesh(core_axis_name="c", subcore_axis_name="s")
    it8 = item_table.T.reshape(2, 8, N_ROWS)
    ut8 = user_table.T.reshape(2, 8, N_ROWS)

    fmt = pl.kernel(
        _fmt_body,
        out_type=(jax.ShapeDtypeStruct((N_ROWS * DIM,), jnp.float32),
                  jax.ShapeDtypeStruct((N_ROWS * DIM,), jnp.float32)),
        mesh=mesh,
        compiler_params=pltpu.CompilerParams(
            needs_layout_passes=False, use_tc_tiling_on_sc=True),
        scratch_types=(
            [pltpu.VMEM((2, 8, CW), jnp.float32), pltpu.VMEM((CW * DIM,), jnp.float32)] * 4
            + [pltpu.SemaphoreType.DMA] * 8),
    )
    itF, utF = fmt(it8, ut8)
    t0 = NCHK * CW
    itF = lax.dynamic_update_slice(itF, item_table[t0:].reshape(-1), (t0 * DIM,))
    utF = lax.dynamic_update_slice(utF, user_table[t0:].reshape(-1), (t0 * DIM,))

    item_idx_flat = item_indices.reshape(B * L)
    w16 = jnp.broadcast_to(W.reshape(DIM, 1), (DIM, DIM))
    bias16 = jnp.broadcast_to(b, (DIM,))
    uidl = jnp.asarray(np.arange(CI, dtype=np.int32) // L)

    gmf = pl.kernel(
        _gmf_body,
        out_type=jax.ShapeDtypeStruct((B * L,), jnp.float32),
        mesh=mesh,
        compiler_params=pltpu.CompilerParams(
            needs_layout_passes=False, use_tc_tiling_on_sc=False),
        scratch_types=(
            [pltpu.VMEM((CI,), jnp.int32),
             pltpu.VMEM((DIM, DIM), jnp.float32),
             pltpu.VMEM((DIM,), jnp.float32)]
            + [pltpu.VMEM((CU,), jnp.int32),
               pltpu.VMEM((CI,), jnp.int32),
               pltpu.VMEM((CU, DIM), jnp.float32),
               pltpu.VMEM((CI, DIM), jnp.float32),
               pltpu.VMEM((CI,), jnp.float32)] * 2
            + [pltpu.SemaphoreType.DMA] * 4),
    )
    out = gmf(user_indices, item_idx_flat,
              utF.reshape(N_ROWS, DIM), itF.reshape(N_ROWS, DIM),
              w16, bias16, uidl)
    return out.reshape(B, L, 1)


def kernel(user_indices, item_indices, user_table, item_table, W, b):
    return _run(user_indices, item_indices, user_table, item_table, W, b)


# speedup vs baseline: 2.4793x; 1.1173x over previous
import jax
import jax.numpy as jnp
import numpy as np
from jax import lax
from jax.experimental import pallas as pl
from jax.experimental.pallas import tpu as pltpu
from jax.experimental.pallas import tpu_sc as plsc

N_ROWS = 1000000
DIM = 16
B = 16384
L = 50

NC = 2
NS = 16
NW = NC * NS

CW = 512
NCHK = 999936 // CW
TAIL = N_ROWS - NCHK * CW
NJ = 62

UPW = B // NW
CU = 64
NCHUNK = UPW // CU
CI = CU * L
NG = CI // DIM


def _transpose_chunk(inv, outv, ncols, iota16):
    sc16 = iota16 * DIM

    def blk(c32, _):
        for t in range(2):
            cbase = (c32 * 2 + t) * DIM
            for b in range(2):
                for d8 in range(8):
                    vec = inv[b, d8, pl.ds(cbase, DIM)]
                    idx = sc16 + (cbase * DIM + b * 8 + d8)
                    plsc.store_scatter(outv, [idx], vec)
        return 0
    lax.fori_loop(0, ncols // (2 * DIM), blk, 0)


def _fmt_body(it8, ut8, itF, utF,
              iiA, ioA, uiA, uoA, iiB, ioB, uiB, uoB,
              s_iiA, s_ioA, s_uiA, s_uoA, s_iiB, s_ioB, s_uiB, s_uoB):
    wid = lax.axis_index("s") * NC + lax.axis_index("c")
    iota16 = lax.iota(jnp.int32, DIM)

    def cid_of(j):
        return jnp.minimum(wid + j * NW, NCHK - 1)

    def issue_in(tab, inv, sem, j):
        c0 = pl.multiple_of(cid_of(j) * CW, CW)
        return pltpu.async_copy(tab.at[:, :, pl.ds(c0, CW)], inv, sem)

    streams = (
        (it8, iiA, ioA, s_iiA, s_ioA, itF, 0),
        (ut8, uiA, uoA, s_uiA, s_uoA, utF, 0),
        (it8, iiB, ioB, s_iiB, s_ioB, itF, 1),
        (ut8, uiB, uoB, s_uiB, s_uoB, utF, 1),
    )
    for tab, inv, _, si, _, _, par in streams:
        issue_in(tab, inv, si, par)

    def body(k2, _):
        for tab, inv, outv, si, so, dstF, par in streams:
            j = 2 * k2 + par
            pltpu.make_async_copy(tab.at[:, :, pl.ds(0, CW)], inv, si).wait()
            @pl.when(k2 > 0)
            def _():
                pltpu.make_async_copy(outv, dstF.at[pl.ds(0, CW * DIM)], so).wait()
            _transpose_chunk(inv, outv, CW, iota16)
            o0 = pl.multiple_of(cid_of(j) * (CW * DIM), CW * DIM)
            pltpu.async_copy(outv, dstF.at[pl.ds(o0, CW * DIM)], so)
            issue_in(tab, inv, si, j + 2)
        return 0

    lax.fori_loop(0, NJ // 2, body, 0)
    for tab, inv, outv, si, so, dstF, par in streams:
        pltpu.make_async_copy(tab.at[:, :, pl.ds(0, CW)], inv, si).wait()
        pltpu.make_async_copy(outv, dstF.at[pl.ds(0, CW * DIM)], so).wait()


def _gmf_body(uidx_hbm, iidx_hbm, utab_hbm, itab_hbm, w_hbm, bias_hbm, uidl_hbm,
              out_hbm,
              uidl_v, wv, biasv,
              uidx_v0, iidx_v0, urows_v0, irows_v0, out_v0,
              uidx_v1, iidx_v1, urows_v1, irows_v1, out_v1,
              sem_g0, sem_g1, sem_o0, sem_o1):
    wid = lax.axis_index("s") * NC + lax.axis_index("c")

    pltpu.sync_copy(w_hbm, wv)
    pltpu.sync_copy(bias_hbm, biasv)
    pltpu.sync_copy(uidl_hbm, uidl_v)

    iota = lax.iota(jnp.int32, DIM)
    fds = [jnp.full((DIM,), d, dtype=jnp.int32) for d in range(DIM)]
    wsplat = [wv[d, :] for d in range(DIM)]
    bias_vec = biasv[...]

    bufs = [
        (uidx_v0, iidx_v0, urows_v0, irows_v0, out_v0, sem_g0, sem_o0),
        (uidx_v1, iidx_v1, urows_v1, irows_v1, out_v1, sem_g1, sem_o1),
    ]

    def issue(c):
        uidx_v, iidx_v, urows_v, irows_v, _, sg, _ = bufs[c % 2]
        ubase = wid * UPW + c * CU
        pltpu.sync_copy(uidx_hbm.at[pl.ds(ubase, CU)], uidx_v)
        pltpu.sync_copy(iidx_hbm.at[pl.ds(ubase * L, CI)], iidx_v)
        cu = pltpu.async_copy(utab_hbm.at[uidx_v], urows_v, sg)
        ci = pltpu.async_copy(itab_hbm.at[iidx_v], irows_v, sg)
        return cu, ci

    pending = issue(0)
    out_pending = [None, None]
    for c in range(NCHUNK):
        slot = c % 2
        _, _, urows_v, irows_v, out_v, _, so = bufs[slot]
        nxt = issue(c + 1) if c + 1 < NCHUNK else None
        pending[0].wait()
        pending[1].wait()
        if out_pending[slot] is not None:
            out_pending[slot].wait()
            out_pending[slot] = None

        def group_body(g, _, irows_v=irows_v, urows_v=urows_v, out_v=out_v):
            gb = pl.multiple_of(g * DIM, DIM)
            pos = gb + iota
            uid = uidl_v[pl.ds(gb, DIM)]
            acc = bias_vec
            for d in range(DIM):
                col = plsc.load_gather(irows_v, [pos, fds[d]])
                pd = plsc.load_gather(urows_v, [uid, fds[d]])
                acc = acc + col * pd * wsplat[d]
            out_v[pl.ds(gb, DIM)] = acc
            return 0

        lax.fori_loop(0, NG, group_body, 0)
        ibase = wid * UPW * L + c * CI
        out_pending[slot] = pltpu.async_copy(
            out_v, out_hbm.at[pl.ds(ibase, CI)], so)
        pending = nxt

    for slot in range(2):
        if out_pending[slot] is not None:
            out_pending[slot].wait()


@jax.jit
def _run(user_indices, item_indices, user_table, item_table, W, b):
    mesh = plsc.VectorSubcoreMesh(core_axis_name="c", subcore_axis_name="s")
    it8 = item_table.T.reshape(2, 8, N_ROWS)
    ut8 = user_table.T.reshape(2, 8, N_ROWS)

    fmt = pl.kernel(
        _fmt_body,
        out_type=(jax.ShapeDtypeStruct((N_ROWS * DIM,), jnp.float32),
                  jax.ShapeDtypeStruct((N_ROWS * DIM,), jnp.float32)),
        mesh=mesh,
        compiler_params=pltpu.CompilerParams(
            needs_layout_passes=False, use_tc_tiling_on_sc=True),
        scratch_types=(
            [pltpu.VMEM((2, 8, CW), jnp.float32), pltpu.VMEM((CW * DIM,), jnp.float32)] * 4
            + [pltpu.SemaphoreType.DMA] * 8),
    )
    itF, utF = fmt(it8, ut8)
    t0 = NCHK * CW
    itF = lax.dynamic_update_slice(itF, item_table[t0:].reshape(-1), (t0 * DIM,))
    utF = lax.dynamic_update_slice(utF, user_table[t0:].reshape(-1), (t0 * DIM,))

    item_idx_flat = item_indices.reshape(B * L)
    w16 = jnp.broadcast_to(W.reshape(DIM, 1), (DIM, DIM))
    bias16 = jnp.broadcast_to(b, (DIM,))
    uidl = jnp.asarray(np.arange(CI, dtype=np.int32) // L)

    gmf = pl.kernel(
        _gmf_body,
        out_type=jax.ShapeDtypeStruct((B * L,), jnp.float32),
        mesh=mesh,
        compiler_params=pltpu.CompilerParams(
            needs_layout_passes=False, use_tc_tiling_on_sc=False),
        scratch_types=(
            [pltpu.VMEM((CI,), jnp.int32),
             pltpu.VMEM((DIM, DIM), jnp.float32),
             pltpu.VMEM((DIM,), jnp.float32)]
            + [pltpu.VMEM((CU,), jnp.int32),
               pltpu.VMEM((CI,), jnp.int32),
               pltpu.VMEM((CU, DIM), jnp.float32),
               pltpu.VMEM((CI, DIM), jnp.float32),
               pltpu.VMEM((CI,), jnp.float32)] * 2
            + [pltpu.SemaphoreType.DMA] * 4),
    )
    out = gmf(user_indices, item_idx_flat,
              utF.reshape(N_ROWS, DIM), itF.reshape(N_ROWS, DIM),
              w16, bias16, uidl)
    return out.reshape(B, L, 1)


def kernel(user_indices, item_indices, user_table, item_table, W, b):
    return _run(user_indices, item_indices, user_table, item_table, W, b)
